# speedup vs baseline: 7.6261x; 7.6261x over previous
import functools

import jax
import jax.numpy as jnp
from jax import lax
from jax.experimental import pallas as pl
from jax.experimental.pallas import tpu as pltpu
from jax.experimental.pallas import tpu_sc as plsc

N = 10000
E = 320000
D = 128

NC = 2
NS = 16
NW = NC * NS
EPW = E // NW
CHUNK = 80
NCHUNK = EPW // CHUNK
NPAD = 10240
STRIPE = NPAD // NS

_MESH = plsc.VectorSubcoreMesh(core_axis_name="c", subcore_axis_name="s")


@functools.partial(
    pl.kernel,
    out_type=jax.ShapeDtypeStruct((NC, 2, NPAD), jnp.float32),
    mesh=_MESH,
    scratch_types=[
        pltpu.VMEM((NCHUNK, CHUNK), jnp.int32),
        pltpu.VMEM((NCHUNK, CHUNK), jnp.int32),
        pltpu.VMEM((CHUNK,), jnp.float32),
        pltpu.VMEM_SHARED((NPAD,), jnp.float32),
        pltpu.VMEM_SHARED((NPAD,), jnp.float32),
    ],
)
def _sc_degrees(src_hbm, dst_hbm, ones_hbm, zeros1_hbm, out_hbm,
                src_v, dst_v, ones_v, acc_out, acc_in):
    cid = lax.axis_index("c")
    sid = lax.axis_index("s")
    wid = sid * NC + cid
    base = sid * STRIPE
    pltpu.sync_copy(zeros1_hbm, acc_out.at[pl.ds(base, STRIPE)])
    pltpu.sync_copy(zeros1_hbm, acc_in.at[pl.ds(base, STRIPE)])
    pltpu.sync_copy(src_hbm.at[wid], src_v)
    pltpu.sync_copy(dst_hbm.at[wid], dst_v)
    pltpu.sync_copy(ones_hbm, ones_v)
    plsc.subcore_barrier()

    def body(j, carry):
        pltpu.sync_copy(ones_v, acc_out.at[src_v.at[j]], add=True)
        pltpu.sync_copy(ones_v, acc_in.at[dst_v.at[j]], add=True)
        return carry

    lax.fori_loop(0, NCHUNK, body, 0)
    plsc.subcore_barrier()
    pltpu.sync_copy(acc_out.at[pl.ds(base, STRIPE)],
                    out_hbm.at[cid, 0, pl.ds(base, STRIPE)])
    pltpu.sync_copy(acc_in.at[pl.ds(base, STRIPE)],
                    out_hbm.at[cid, 1, pl.ds(base, STRIPE)])


@functools.partial(
    pl.kernel,
    out_type=jax.ShapeDtypeStruct((NC, NPAD, D), jnp.float32),
    mesh=_MESH,
    scratch_types=[
        pltpu.VMEM((NCHUNK, CHUNK), jnp.int32),
        pltpu.VMEM((NCHUNK, CHUNK), jnp.int32),
        pltpu.VMEM((CHUNK, D), jnp.float32),
        pltpu.VMEM_SHARED((NPAD, D), jnp.float32),
        pltpu.SemaphoreType.DMA,
    ],
)
def _sc_scatter_rows(src_hbm, dst_hbm, xs_hbm, zeros2_hbm, out_hbm,
                     src_v, dst_v, rows_v, acc, sem):
    cid = lax.axis_index("c")
    sid = lax.axis_index("s")
    wid = sid * NC + cid
    base = sid * STRIPE
    pltpu.sync_copy(zeros2_hbm, acc.at[pl.ds(base, STRIPE)])
    pltpu.sync_copy(src_hbm.at[wid], src_v)
    pltpu.sync_copy(dst_hbm.at[wid], dst_v)
    plsc.subcore_barrier()

    def body(j, carry):
        pltpu.async_copy(xs_hbm.at[src_v.at[j]], rows_v, sem).wait()
        pltpu.sync_copy(rows_v, acc.at[dst_v.at[j]], add=True)
        return carry

    lax.fori_loop(0, NCHUNK, body, 0)
    plsc.subcore_barrier()
    pltpu.sync_copy(acc.at[pl.ds(base, STRIPE)],
                    out_hbm.at[cid, pl.ds(base, STRIPE)])


def _tc_norms_body(deg_ref, h_ref, xs_ref, nout_ref, nin_ref):
    out_deg = deg_ref[0, 0] + deg_ref[1, 0]
    in_deg = deg_ref[0, 1] + deg_ref[1, 1]
    norm_out = lax.rsqrt(jnp.maximum(out_deg, 1.0))[:N]
    norm_in = lax.rsqrt(jnp.maximum(in_deg, 1.0))[:N]
    xs_ref[...] = h_ref[...] * norm_out
    nout_ref[...] = norm_out
    nin_ref[...] = norm_in


def _tc_layer1_body(aggp_ref, nin_ref, nout_ref, w_ref, b_ref,
                    xs2_ref, skip_ref):
    agg = (aggp_ref[0] + aggp_ref[1])[:N] * nin_ref[...]
    x = jnp.dot(agg, w_ref[...], preferred_element_type=jnp.float32)
    x = jnp.maximum(x + b_ref[...], 0.0)
    skip_ref[...] = jnp.sum(x, axis=0, keepdims=True) * (1.0 / N)
    xs2_ref[...] = x * nout_ref[...]


def _tc_layer2_body(aggp_ref, nin_ref, w_ref, b_ref, skip1_ref, out_ref):
    agg = (aggp_ref[0] + aggp_ref[1])[:N] * nin_ref[...]
    x = jnp.dot(agg, w_ref[...], preferred_element_type=jnp.float32)
    x = jnp.maximum(x + b_ref[...], 0.0)
    out_ref[...] = skip1_ref[...] + 2.0 * (jnp.sum(x, axis=0, keepdims=True)
                                           * (1.0 / N))


_tc_norms = pl.pallas_call(
    _tc_norms_body,
    out_shape=(
        jax.ShapeDtypeStruct((N, D), jnp.float32),
        jax.ShapeDtypeStruct((N, 1), jnp.float32),
        jax.ShapeDtypeStruct((N, 1), jnp.float32),
    ),
)

_tc_layer1 = pl.pallas_call(
    _tc_layer1_body,
    out_shape=(
        jax.ShapeDtypeStruct((N, D), jnp.float32),
        jax.ShapeDtypeStruct((1, D), jnp.float32),
    ),
)

_tc_layer2 = pl.pallas_call(
    _tc_layer2_body,
    out_shape=jax.ShapeDtypeStruct((1, D), jnp.float32),
)


@jax.jit
def kernel(h, edge_index, W1, b1, W2, b2):
    src3 = edge_index[0].reshape(NW, NCHUNK, CHUNK)
    dst3 = edge_index[1].reshape(NW, NCHUNK, CHUNK)
    ones = jnp.ones((CHUNK,), jnp.float32)
    zeros1 = jnp.zeros((STRIPE,), jnp.float32)
    zeros2 = jnp.zeros((STRIPE, D), jnp.float32)

    deg = _sc_degrees(src3, dst3, ones, zeros1)
    deg4 = deg.reshape(NC, 2, NPAD, 1)
    xs1, norm_out, norm_in = _tc_norms(deg4, h)

    agg1 = _sc_scatter_rows(src3, dst3, xs1, zeros2)
    xs2, skip1 = _tc_layer1(agg1, norm_in, norm_out, W1, b1.reshape(1, D))

    agg2 = _sc_scatter_rows(src3, dst3, xs2, zeros2)
    return _tc_layer2(agg2, norm_in, W2, b2.reshape(1, D), skip1)


# speedup vs baseline: 10.3826x; 1.3615x over previous
import functools

import jax
import jax.numpy as jnp
from jax import lax
from jax.experimental import pallas as pl
from jax.experimental.pallas import tpu as pltpu
from jax.experimental.pallas import tpu_sc as plsc

N = 10000
E = 320000
D = 128

NC = 2
NS = 16
NW = NC * NS
EPW = E // NW
CHUNK = 80
NCHUNK = EPW // CHUNK
NBLK = 5
BCHUNK = NCHUNK // NBLK
NPAD = 10240
STRIPE = NPAD // NS
ZROWS = STRIPE // 4

_MESH = plsc.VectorSubcoreMesh(core_axis_name="c", subcore_axis_name="s")


@functools.partial(
    pl.kernel,
    out_type=jax.ShapeDtypeStruct((NC, 2, NPAD), jnp.float32),
    mesh=_MESH,
    scratch_types=[
        pltpu.VMEM((NCHUNK, CHUNK), jnp.int32),
        pltpu.VMEM((NCHUNK, CHUNK), jnp.int32),
        pltpu.VMEM((CHUNK,), jnp.float32),
        pltpu.VMEM_SHARED((NPAD,), jnp.float32),
        pltpu.VMEM_SHARED((NPAD,), jnp.float32),
    ],
)
def _sc_degrees(src_hbm, dst_hbm, ones_hbm, zeros1_hbm, out_hbm,
                src_v, dst_v, ones_v, acc_out, acc_in):
    cid = lax.axis_index("c")
    sid = lax.axis_index("s")
    wid = sid * NC + cid
    base = sid * STRIPE
    pltpu.sync_copy(zeros1_hbm, acc_out.at[pl.ds(base, STRIPE)])
    pltpu.sync_copy(zeros1_hbm, acc_in.at[pl.ds(base, STRIPE)])
    pltpu.sync_copy(src_hbm.at[wid], src_v)
    pltpu.sync_copy(dst_hbm.at[wid], dst_v)
    pltpu.sync_copy(ones_hbm, ones_v)
    plsc.subcore_barrier()

    def body(j, carry):
        pltpu.sync_copy(ones_v, acc_out.at[src_v.at[j]], add=True)
        pltpu.sync_copy(ones_v, acc_in.at[dst_v.at[j]], add=True)
        return carry

    lax.fori_loop(0, NCHUNK, body, 0)
    plsc.subcore_barrier()
    pltpu.sync_copy(acc_out.at[pl.ds(base, STRIPE)],
                    out_hbm.at[cid, 0, pl.ds(base, STRIPE)])
    pltpu.sync_copy(acc_in.at[pl.ds(base, STRIPE)],
                    out_hbm.at[cid, 1, pl.ds(base, STRIPE)])


@functools.partial(
    pl.kernel,
    out_type=jax.ShapeDtypeStruct((NC, NPAD, D), jnp.float32),
    mesh=_MESH,
    scratch_types=[
        pltpu.VMEM((BCHUNK, CHUNK), jnp.int32),
        pltpu.VMEM((BCHUNK, CHUNK), jnp.int32),
        pltpu.VMEM((2, CHUNK, D), jnp.float32),
        pltpu.VMEM_SHARED((NPAD, D), jnp.float32),
        pltpu.SemaphoreType.DMA,
        pltpu.SemaphoreType.DMA,
    ],
)
def _sc_scatter_rows(src_hbm, dst_hbm, xs_hbm, zeros2_hbm, out_hbm,
                     src_v, dst_v, rows_v, acc, sem0, sem1):
    cid = lax.axis_index("c")
    sid = lax.axis_index("s")
    wid = sid * NC + cid
    base = sid * STRIPE

    def zbody(i, carry):
        pltpu.sync_copy(zeros2_hbm, acc.at[pl.ds(base + i * ZROWS, ZROWS)])
        return carry

    lax.fori_loop(0, 4, zbody, 0)
    plsc.subcore_barrier()

    def blk_body(blk, carry):
        pltpu.sync_copy(src_hbm.at[wid, blk], src_v)
        pltpu.sync_copy(dst_hbm.at[wid, blk], dst_v)
        pltpu.async_copy(xs_hbm.at[src_v.at[0]], rows_v.at[0], sem0)

        def body(k, c):
            j = 2 * k
            pltpu.async_copy(xs_hbm.at[src_v.at[j + 1]], rows_v.at[1], sem1)
            pltpu.make_async_copy(xs_hbm.at[src_v.at[j]],
                                  rows_v.at[0], sem0).wait()
            pltpu.sync_copy(rows_v.at[0], acc.at[dst_v.at[j]], add=True)
            pltpu.async_copy(xs_hbm.at[src_v.at[j + 2]], rows_v.at[0], sem0)
            pltpu.make_async_copy(xs_hbm.at[src_v.at[j + 1]],
                                  rows_v.at[1], sem1).wait()
            pltpu.sync_copy(rows_v.at[1], acc.at[dst_v.at[j + 1]], add=True)
            return c

        lax.fori_loop(0, (BCHUNK - 1) // 2, body, 0)
        pltpu.make_async_copy(xs_hbm.at[src_v.at[BCHUNK - 1]],
                              rows_v.at[0], sem0).wait()
        pltpu.sync_copy(rows_v.at[0], acc.at[dst_v.at[BCHUNK - 1]], add=True)
        return carry

    lax.fori_loop(0, NBLK, blk_body, 0)
    plsc.subcore_barrier()
    pltpu.sync_copy(acc.at[pl.ds(base, STRIPE)],
                    out_hbm.at[cid, pl.ds(base, STRIPE)])


def _tc_norms_body(deg_ref, h_ref, xs_ref, nout_ref, nin_ref):
    out_deg = deg_ref[0, 0] + deg_ref[1, 0]
    in_deg = deg_ref[0, 1] + deg_ref[1, 1]
    norm_out = lax.rsqrt(jnp.maximum(out_deg, 1.0))[:N]
    norm_in = lax.rsqrt(jnp.maximum(in_deg, 1.0))[:N]
    xs_ref[...] = h_ref[...] * norm_out
    nout_ref[...] = norm_out
    nin_ref[...] = norm_in


def _tc_layer1_body(aggp_ref, nin_ref, nout_ref, w_ref, b_ref,
                    xs2_ref, skip_ref):
    agg = (aggp_ref[0] + aggp_ref[1])[:N] * nin_ref[...]
    x = jnp.dot(agg, w_ref[...], preferred_element_type=jnp.float32)
    x = jnp.maximum(x + b_ref[...], 0.0)
    skip_ref[...] = jnp.sum(x, axis=0, keepdims=True) * (1.0 / N)
    xs2_ref[...] = x * nout_ref[...]


def _tc_layer2_body(aggp_ref, nin_ref, w_ref, b_ref, skip1_ref, out_ref):
    agg = (aggp_ref[0] + aggp_ref[1])[:N] * nin_ref[...]
    x = jnp.dot(agg, w_ref[...], preferred_element_type=jnp.float32)
    x = jnp.maximum(x + b_ref[...], 0.0)
    out_ref[...] = skip1_ref[...] + 2.0 * (jnp.sum(x, axis=0, keepdims=True)
                                           * (1.0 / N))


_tc_norms = pl.pallas_call(
    _tc_norms_body,
    out_shape=(
        jax.ShapeDtypeStruct((N, D), jnp.float32),
        jax.ShapeDtypeStruct((N, 1), jnp.float32),
        jax.ShapeDtypeStruct((N, 1), jnp.float32),
    ),
)

_tc_layer1 = pl.pallas_call(
    _tc_layer1_body,
    out_shape=(
        jax.ShapeDtypeStruct((N, D), jnp.float32),
        jax.ShapeDtypeStruct((1, D), jnp.float32),
    ),
)

_tc_layer2 = pl.pallas_call(
    _tc_layer2_body,
    out_shape=jax.ShapeDtypeStruct((1, D), jnp.float32),
)


@jax.jit
def kernel(h, edge_index, W1, b1, W2, b2):
    src3 = edge_index[0].reshape(NW, NCHUNK, CHUNK)
    dst3 = edge_index[1].reshape(NW, NCHUNK, CHUNK)
    src4 = src3.reshape(NW, NBLK, BCHUNK, CHUNK)
    dst4 = dst3.reshape(NW, NBLK, BCHUNK, CHUNK)
    ones = jnp.ones((CHUNK,), jnp.float32)
    zeros1 = jnp.zeros((STRIPE,), jnp.float32)
    zeros2 = jnp.zeros((ZROWS, D), jnp.float32)

    deg = _sc_degrees(src3, dst3, ones, zeros1)
    deg4 = deg.reshape(NC, 2, NPAD, 1)
    xs1, norm_out, norm_in = _tc_norms(deg4, h)

    agg1 = _sc_scatter_rows(src4, dst4, xs1, zeros2)
    xs2, skip1 = _tc_layer1(agg1, norm_in, norm_out, W1, b1.reshape(1, D))

    agg2 = _sc_scatter_rows(src4, dst4, xs2, zeros2)
    return _tc_layer2(agg2, norm_in, W2, b2.reshape(1, D), skip1)
